# baseline (speedup 1.0000x reference)
import jax
import jax.numpy as jnp
from jax.experimental import pallas as pl
from jax.experimental.pallas import tpu as pltpu

_LN_EPS = 1e-5
_HEADS = 4
_DH = 8
_W = 8
_DIM = 16

_PAR = pltpu.CompilerParams(dimension_semantics=("parallel",))


def _silu(v):
    return v * jax.nn.sigmoid(v)


def _rms_center(x):
    d = x.shape[-1]
    avg = jnp.full((d, d), 1.0 / d, jnp.float32)
    mu = jnp.dot(x, avg, preferred_element_type=jnp.float32)
    xc = x - mu
    var = jnp.dot(xc * xc, avg, preferred_element_type=jnp.float32)
    return xc * jax.lax.rsqrt(var + _LN_EPS)


def _conv3x3(x3, m):
    B, H, C = x3.shape
    z = jnp.zeros((B, 1, C), x3.dtype)
    dn = jnp.concatenate([z, x3[:, :H - 1, :]], axis=1)
    up = jnp.concatenate([x3[:, 1:, :], z], axis=1)
    f = lambda a, w: jnp.dot(a.reshape(B * H, C), w,
                             preferred_element_type=jnp.float32)
    out = f(dn, m[0]) + f(x3, m[1]) + f(up, m[2])
    return out.reshape(B, H, out.shape[-1])


def _net_body(p_ref, w0, b0, we, be, mdw, bdw, wpr, bpr, m1, b1c, w2c, b2c,
              wqkv, bqkv, wof, bo, w1f, bf1f, w2f, bf2, w3, b3c,
              m4, b4, wl, bl, wh, bh, o_ref):
    B, H, P216 = p_ref.shape
    x0 = p_ref[...].reshape(B * H, P216)
    ys = _silu(jnp.dot(x0, w0[...], preferred_element_type=jnp.float32)
               + b0[...])
    e = _silu(jnp.dot(ys, we[...], preferred_element_type=jnp.float32)
              + be[...])
    dw = _conv3x3(e.reshape(B, H, 128), mdw[...])
    ydw = _silu(dw.reshape(B * H, 128) + bdw[...])
    y0 = (jnp.dot(ydw, wpr[...], preferred_element_type=jnp.float32)
          + bpr[...] + ys)
    c1 = _conv3x3(y0.reshape(B, H, 64), m1[...]).reshape(B * H, 64)
    y1 = _silu(c1 + b1c[...])
    c2 = _silu(jnp.dot(y1, w2c[...], preferred_element_type=jnp.float32)
               + b2c[...])

    c26 = c2.reshape(B, 4, 2, 4, 2, _DIM)
    xt = jnp.transpose(c26, (0, 2, 4, 1, 3, 5))
    n_tok = B * 4 * 16
    S = 16
    hd_all = _HEADS * _DH
    x = xt.reshape(n_tok, _DIM)

    z2 = _rms_center(x)
    qkv = jnp.dot(z2, wqkv[...], preferred_element_type=jnp.float32) \
        + bqkv[...]
    ones_col = jnp.full((S, 1), 1.0, jnp.float32)
    ohs = []
    for h in range(_HEADS):
        qh = qkv[:, h * _DH:(h + 1) * _DH].reshape(B * 4, S, _DH)
        kh = qkv[:, hd_all + h * _DH:hd_all + (h + 1) * _DH].reshape(
            B * 4, S, _DH)
        vh = qkv[:, 2 * hd_all + h * _DH:2 * hd_all + (h + 1) * _DH].reshape(
            B * 4, S, _DH)
        s = jax.lax.dot_general(
            qh, kh, (((2,), (2,)), ((0,), (0,))),
            preferred_element_type=jnp.float32)
        es = jnp.exp(s)
        denom = jnp.dot(es.reshape(n_tok, S), ones_col,
                        preferred_element_type=jnp.float32)
        oh = jax.lax.dot_general(
            es, vh, (((2,), (1,)), ((0,), (0,))),
            preferred_element_type=jnp.float32)
        ohs.append(oh.reshape(n_tok, _DH) / denom)
    x = x + bo[...] + jnp.dot(jnp.concatenate(ohs, axis=1), wof[...],
                              preferred_element_type=jnp.float32)
    hid = _silu(jnp.dot(_rms_center(x), w1f[...],
                        preferred_element_type=jnp.float32) + bf1f[...])
    x = x + jnp.dot(hid, w2f[...], preferred_element_type=jnp.float32) \
        + bf2[...]
    y = _silu(jnp.dot(x, w3[...], preferred_element_type=jnp.float32)
              + b3c[...])

    y6 = y.reshape(B, 2, 2, 4, 4, 8)
    c3sp = jnp.transpose(y6, (0, 3, 1, 4, 2, 5)).reshape(B, H, _W, 8)
    cat3 = jnp.concatenate([c3sp, y0.reshape(B, H, _W, 8)],
                           axis=-1).reshape(B, H, _W * 16)

    c4 = _conv3x3(cat3, m4[...]).reshape(B * H, 64)
    yt = _silu(c4 + b4[...])
    zt = _silu(jnp.dot(yt, wl[...], preferred_element_type=jnp.float32)
               + bl[...])
    pooled = jnp.mean(zt.reshape(B, H, 128), axis=1)
    logits = jnp.dot(pooled, wh[...], preferred_element_type=jnp.float32) \
        + bh[...]
    mx = jnp.max(logits, axis=-1, keepdims=True)
    ex = jnp.exp(logits - mx)
    o_ref[...] = ex / jnp.sum(ex, axis=-1, keepdims=True)


def _blk(shape, b):
    nd = len(shape)
    return pl.BlockSpec((b,) + tuple(shape[1:]),
                        lambda i: (i,) + (0,) * (nd - 1))


def _full(shape):
    nd = len(shape)
    return pl.BlockSpec(tuple(shape), lambda i: (0,) * nd)


def _pick_b(n, want):
    b = min(want, n)
    while n % b:
        b //= 2
    return b


def kernel(x, sd, su, stem_w, stem_a, stem_b, exp_w, exp_a, exp_b, dw_m, dw_a,
           dw_b, proj_w, proj_a, proj_b, c1_m, c1_a, c1_b, c2_w, c2_a, c2_b,
           c3_w, c3_a, c3_b, c4_m, c4_a, c4_b, last_w, last_a, last_b, head_w,
           head_b, l0_g1, l0_b1, l0_wq, l0_wk, l0_wv, l0_wo, l0_bo, l0_g2,
           l0_b2, l0_w1, l0_bf1, l0_w2, l0_bf2):
    n = x.shape[0]
    xh = jnp.transpose(x, (0, 2, 3, 1))

    xp = jnp.pad(xh, ((0, 0), (1, 1), (1, 1), (0, 0)))
    taps = [xp[:, i:i + 16:2, j:j + 16:2, :] for i in range(3) for j in range(3)]
    patches = jnp.concatenate(taps, axis=-1).reshape(n, _W, _W * 27)

    w0 = stem_w * stem_a
    we = exp_w * exp_a
    mdw = dw_m * dw_a
    wpr = proj_w * proj_a
    m1 = c1_m * c1_a
    w2c = c2_w * c2_a
    w3 = c3_w * c3_a
    m4 = c4_m * c4_a
    wl = last_w * last_a

    scale = _DH ** -0.5
    wqc = jnp.transpose(l0_wq, (1, 0, 2)).reshape(_DIM, _HEADS * _DH) * scale
    wkc = jnp.transpose(l0_wk, (1, 0, 2)).reshape(_DIM, _HEADS * _DH)
    wvc = jnp.transpose(l0_wv, (1, 0, 2)).reshape(_DIM, _HEADS * _DH)
    wqkv_c = jnp.concatenate([wqc, wkc, wvc], axis=1)
    wqkv = wqkv_c * l0_g1.reshape(_DIM, 1)
    bqkv = jnp.dot(l0_b1, wqkv_c)
    wof = l0_wo.reshape(_HEADS * _DH, _DIM)
    w1f = l0_w1 * l0_g2.reshape(_DIM, 1)
    bf1f = l0_bf1 + jnp.dot(l0_b2, l0_w1)

    ops = (w0, stem_b, we, exp_b, mdw, dw_b, wpr, proj_b, m1, c1_b, w2c, c2_b,
           wqkv, bqkv, wof, l0_bo, w1f, bf1f, l0_w2, l0_bf2, w3, c3_b,
           m4, c4_b, wl, last_b, head_w, head_b)

    bb = _pick_b(n, 128)
    probs = pl.pallas_call(
        _net_body, grid=(n // bb,),
        in_specs=[_blk(patches.shape, bb)] + [_full(a.shape) for a in ops],
        out_specs=_blk((n, 5), bb),
        out_shape=jax.ShapeDtypeStruct((n, 5), jnp.float32),
        compiler_params=_PAR,
    )(patches, *ops)
    return probs


# speedup vs baseline: 1.1819x; 1.1819x over previous
import jax
import jax.numpy as jnp
from jax.experimental import pallas as pl
from jax.experimental.pallas import tpu as pltpu

_LN_EPS = 1e-5
_HEADS = 4
_DH = 8
_W = 8
_DIM = 16

_PAR = pltpu.CompilerParams(dimension_semantics=("parallel",))


def _silu(v):
    return v * jax.nn.sigmoid(v)


def _ln(x, g, b):
    mu = jnp.mean(x, axis=-1, keepdims=True)
    xc = x - mu
    var = jnp.mean(xc * xc, axis=-1, keepdims=True)
    return xc * jax.lax.rsqrt(var + _LN_EPS) * g + b


def _conv3x3(x3, m):
    B, H, C = x3.shape
    z = jnp.zeros((B, 1, C), x3.dtype)
    dn = jnp.concatenate([z, x3[:, :H - 1, :]], axis=1)
    up = jnp.concatenate([x3[:, 1:, :], z], axis=1)
    f = lambda a, w: jnp.dot(a.reshape(B * H, C), w,
                             preferred_element_type=jnp.float32)
    out = f(dn, m[0]) + f(x3, m[1]) + f(up, m[2])
    return out.reshape(B, H, out.shape[-1])


def _front_body(p_ref, w0, b0, we, be, mdw, bdw, wpr, bpr,
                m1, b1c, w2c, b2c, y0_ref, c2_ref):
    B, H, P = p_ref.shape
    x = p_ref[...].reshape(B * H, P)
    ys = _silu(jnp.dot(x, w0[...], preferred_element_type=jnp.float32)
               + b0[...])
    e = _silu(jnp.dot(ys, we[...], preferred_element_type=jnp.float32)
              + be[...])
    dw = _conv3x3(e.reshape(B, H, 128), mdw[...])
    ydw = _silu(dw.reshape(B * H, 128) + bdw[...])
    y0 = (jnp.dot(ydw, wpr[...], preferred_element_type=jnp.float32)
          + bpr[...] + ys)
    c1 = _conv3x3(y0.reshape(B, H, 64), m1[...]).reshape(B * H, 64)
    y1 = _silu(c1 + b1c[...])
    c2 = _silu(jnp.dot(y1, w2c[...], preferred_element_type=jnp.float32)
               + b2c[...])
    y0_ref[...] = y0.reshape(B, H, 64)
    c2_ref[...] = c2.reshape(B, H, 128)


def _rms_center(x):
    d = x.shape[-1]
    avg = jnp.full((d, d), 1.0 / d, jnp.float32)
    mu = jnp.dot(x, avg, preferred_element_type=jnp.float32)
    xc = x - mu
    var = jnp.dot(xc * xc, avg, preferred_element_type=jnp.float32)
    return xc * jax.lax.rsqrt(var + _LN_EPS)


def _tr_body(t_ref, wqkv, bqkv, woc2, bo, w1f, bf1f, w2f, bf2, w3, b3c,
             o_ref):
    B, P, S, D = t_ref.shape
    n_tok = B * P * S
    hd_all = _HEADS * _DH
    x = t_ref[...].reshape(n_tok, D)
    z = _rms_center(x)
    qkv = jnp.dot(z, wqkv[...], preferred_element_type=jnp.float32) \
        + bqkv[...]
    ones_col = jnp.full((S, 1), 1.0, jnp.float32)
    ohs = []
    for h in range(_HEADS):
        qh = qkv[:, h * _DH:(h + 1) * _DH].reshape(B * P, S, _DH)
        kh = qkv[:, hd_all + h * _DH:hd_all + (h + 1) * _DH].reshape(
            B * P, S, _DH)
        vh = qkv[:, 2 * hd_all + h * _DH:2 * hd_all + (h + 1) * _DH].reshape(
            B * P, S, _DH)
        s = jax.lax.dot_general(
            qh, kh, (((2,), (2,)), ((0,), (0,))),
            preferred_element_type=jnp.float32)
        es = jnp.exp(s)
        denom = jnp.dot(es.reshape(n_tok, S), ones_col,
                        preferred_element_type=jnp.float32)
        oh = jax.lax.dot_general(
            es, vh, (((2,), (1,)), ((0,), (0,))),
            preferred_element_type=jnp.float32)
        ohs.append(oh.reshape(n_tok, _DH) / denom)
    x = x + bo[...] + jnp.dot(jnp.concatenate(ohs, axis=1), woc2[...],
                              preferred_element_type=jnp.float32)
    hid = _silu(jnp.dot(_rms_center(x), w1f[...],
                        preferred_element_type=jnp.float32) + bf1f[...])
    x = x + jnp.dot(hid, w2f[...], preferred_element_type=jnp.float32) + bf2[...]
    y = _silu(jnp.dot(x, w3[...], preferred_element_type=jnp.float32)
              + b3c[...])
    o_ref[...] = y.reshape(B, P, S, 8)


def _tail_body(cat_ref, m4, b4, wl, bl, wh, bh, o_ref):
    B, H, C = cat_ref.shape
    c4 = _conv3x3(cat_ref[...], m4[...]).reshape(B * H, 64)
    y = _silu(c4 + b4[...])
    z = _silu(jnp.dot(y, wl[...], preferred_element_type=jnp.float32)
              + bl[...])
    pooled = jnp.mean(z.reshape(B, H, 128), axis=1)
    logits = jnp.dot(pooled, wh[...], preferred_element_type=jnp.float32) \
        + bh[...]
    mx = jnp.max(logits, axis=-1, keepdims=True)
    ex = jnp.exp(logits - mx)
    o_ref[...] = ex / jnp.sum(ex, axis=-1, keepdims=True)


def _blk(shape, b):
    nd = len(shape)
    return pl.BlockSpec((b,) + tuple(shape[1:]),
                        lambda i: (i,) + (0,) * (nd - 1))


def _full(shape):
    nd = len(shape)
    return pl.BlockSpec(tuple(shape), lambda i: (0,) * nd)


def _pick_b(n, want):
    b = min(want, n)
    while n % b:
        b //= 2
    return b


def kernel(x, sd, su, stem_w, stem_a, stem_b, exp_w, exp_a, exp_b, dw_m, dw_a,
           dw_b, proj_w, proj_a, proj_b, c1_m, c1_a, c1_b, c2_w, c2_a, c2_b,
           c3_w, c3_a, c3_b, c4_m, c4_a, c4_b, last_w, last_a, last_b, head_w,
           head_b, l0_g1, l0_b1, l0_wq, l0_wk, l0_wv, l0_wo, l0_bo, l0_g2,
           l0_b2, l0_w1, l0_bf1, l0_w2, l0_bf2):
    n = x.shape[0]
    xh = jnp.transpose(x, (0, 2, 3, 1))

    xp = jnp.pad(xh, ((0, 0), (1, 1), (1, 1), (0, 0)))
    taps = [xp[:, i:i + 16:2, j:j + 16:2, :] for i in range(3) for j in range(3)]
    patches = jnp.concatenate(taps, axis=-1).reshape(n, _W, _W * 27)

    w0 = stem_w * stem_a
    we = exp_w * exp_a
    mdw = dw_m * dw_a
    wpr = proj_w * proj_a
    m1 = c1_m * c1_a
    w2c = c2_w * c2_a
    w3 = c3_w * c3_a
    m4 = c4_m * c4_a
    wl = last_w * last_a

    bf = _pick_b(n, 256)
    y0, c2 = pl.pallas_call(
        _front_body, grid=(n // bf,),
        in_specs=[_blk(patches.shape, bf)] + [
            _full(a.shape) for a in
            (w0, stem_b, we, exp_b, mdw, dw_b, wpr, proj_b, m1, c1_b, w2c, c2_b)
        ],
        out_specs=(_blk((n, _W, 64), bf), _blk((n, _W, 128), bf)),
        out_shape=(jax.ShapeDtypeStruct((n, _W, 64), jnp.float32),
                   jax.ShapeDtypeStruct((n, _W, 128), jnp.float32)),
        compiler_params=_PAR,
    )(patches, w0, stem_b, we, exp_b, mdw, dw_b, wpr, proj_b, m1, c1_b,
      w2c, c2_b)

    ph = pw = 2
    hh = ww = _W // 2
    z = c2.reshape(n, _W, _W, _DIM)
    t = z.reshape(n, hh, ph, ww, pw, _DIM).transpose(0, 2, 4, 1, 3, 5)
    t = t.reshape(n, ph * pw, hh * ww, _DIM)

    scale = _DH ** -0.5
    wqc = jnp.transpose(l0_wq, (1, 0, 2)).reshape(_DIM, _HEADS * _DH) * scale
    wkc = jnp.transpose(l0_wk, (1, 0, 2)).reshape(_DIM, _HEADS * _DH)
    wvc = jnp.transpose(l0_wv, (1, 0, 2)).reshape(_DIM, _HEADS * _DH)
    wqkv = jnp.concatenate([wqc, wkc, wvc], axis=1)
    wqkv = wqkv * l0_g1.reshape(_DIM, 1)
    bqkv = jnp.dot(l0_b1, jnp.concatenate([wqc, wkc, wvc], axis=1))
    wof = l0_wo.reshape(_HEADS * _DH, _DIM)
    w1f = l0_w1 * l0_g2.reshape(_DIM, 1)
    bf1f = l0_bf1 + jnp.dot(l0_b2, l0_w1)

    bt = _pick_b(n, 128)
    t_out = pl.pallas_call(
        _tr_body, grid=(n // bt,),
        in_specs=[_blk(t.shape, bt)] + [
            _full(a.shape) for a in
            (wqkv, bqkv, wof, l0_bo, w1f, bf1f, l0_w2, l0_bf2, w3, c3_b)
        ],
        out_specs=_blk((n, ph * pw, hh * ww, 8), bt),
        out_shape=jax.ShapeDtypeStruct((n, ph * pw, hh * ww, 8), jnp.float32),
        compiler_params=_PAR,
    )(t, wqkv, bqkv, wof, l0_bo, w1f, bf1f, l0_w2, l0_bf2, w3, c3_b)

    c3s = t_out.reshape(n, ph, pw, hh, ww, 8).transpose(0, 3, 1, 4, 2, 5)
    c3s = c3s.reshape(n, _W, _W, 8)
    y0s = y0.reshape(n, _W, _W, 8)
    cat = jnp.concatenate([c3s, y0s], axis=-1).reshape(n, _W, _W * 16)

    bl_ = _pick_b(n, 256)
    probs = pl.pallas_call(
        _tail_body, grid=(n // bl_,),
        in_specs=[_blk(cat.shape, bl_)] + [
            _full(a.shape) for a in (m4, c4_b, wl, last_b, head_w, head_b)
        ],
        out_specs=_blk((n, 5), bl_),
        out_shape=jax.ShapeDtypeStruct((n, 5), jnp.float32),
        compiler_params=_PAR,
    )(cat, m4, c4_b, wl, last_b, head_w, head_b)
    return probs


# speedup vs baseline: 1.1902x; 1.0071x over previous
import jax
import jax.numpy as jnp
from jax.experimental import pallas as pl
from jax.experimental.pallas import tpu as pltpu

_LN_EPS = 1e-5
_HEADS = 4
_DH = 8
_W = 8
_DIM = 16

_PAR = pltpu.CompilerParams(dimension_semantics=("parallel",))


def _silu(v):
    return v * jax.nn.sigmoid(v)


def _ln(x, g, b):
    mu = jnp.mean(x, axis=-1, keepdims=True)
    xc = x - mu
    var = jnp.mean(xc * xc, axis=-1, keepdims=True)
    return xc * jax.lax.rsqrt(var + _LN_EPS) * g + b


def _conv3x3(x3, m):
    B, H, C = x3.shape
    z = jnp.zeros((B, 1, C), x3.dtype)
    dn = jnp.concatenate([z, x3[:, :H - 1, :]], axis=1)
    up = jnp.concatenate([x3[:, 1:, :], z], axis=1)
    f = lambda a, w: jnp.dot(a.reshape(B * H, C), w,
                             preferred_element_type=jnp.float32)
    out = f(dn, m[0]) + f(x3, m[1]) + f(up, m[2])
    return out.reshape(B, H, out.shape[-1])


def _front_body(p_ref, w0, b0, we, be, mdw, bdw, wpr, bpr,
                m1, b1c, w2c, b2c, y0_ref, c2_ref):
    B, H, P = p_ref.shape
    x = p_ref[...].reshape(B * H, P)
    ys = _silu(jnp.dot(x, w0[...], preferred_element_type=jnp.float32)
               + b0[...])
    e = _silu(jnp.dot(ys, we[...], preferred_element_type=jnp.float32)
              + be[...])
    dw = _conv3x3(e.reshape(B, H, 128), mdw[...])
    ydw = _silu(dw.reshape(B * H, 128) + bdw[...])
    y0 = (jnp.dot(ydw, wpr[...], preferred_element_type=jnp.float32)
          + bpr[...] + ys)
    c1 = _conv3x3(y0.reshape(B, H, 64), m1[...]).reshape(B * H, 64)
    y1 = _silu(c1 + b1c[...])
    c2 = _silu(jnp.dot(y1, w2c[...], preferred_element_type=jnp.float32)
               + b2c[...])
    y0_ref[...] = y0.reshape(B, H, 64)
    c2_ref[...] = c2.reshape(B, H, 128)


def _rms_center(x):
    d = x.shape[-1]
    avg = jnp.full((d, d), 1.0 / d, jnp.float32)
    mu = jnp.dot(x, avg, preferred_element_type=jnp.float32)
    xc = x - mu
    var = jnp.dot(xc * xc, avg, preferred_element_type=jnp.float32)
    return xc * jax.lax.rsqrt(var + _LN_EPS)


def _tr_body(t_ref, wqkv, bqkv, woc2, bo, w1f, bf1f, w2f, bf2, w3, b3c,
             o_ref):
    B, P, S, D = t_ref.shape
    n_tok = B * P * S
    hd_all = _HEADS * _DH
    x = t_ref[...].reshape(n_tok, D)
    z = _rms_center(x)
    qkv = jnp.dot(z, wqkv[...], preferred_element_type=jnp.float32) \
        + bqkv[...]
    ones_col = jnp.full((S, 1), 1.0, jnp.float32)
    ohs = []
    for h in range(_HEADS):
        qh = qkv[:, h * _DH:(h + 1) * _DH].reshape(B * P, S, _DH)
        kh = qkv[:, hd_all + h * _DH:hd_all + (h + 1) * _DH].reshape(
            B * P, S, _DH)
        vh = qkv[:, 2 * hd_all + h * _DH:2 * hd_all + (h + 1) * _DH].reshape(
            B * P, S, _DH)
        s = jax.lax.dot_general(
            qh, kh, (((2,), (2,)), ((0,), (0,))),
            preferred_element_type=jnp.float32)
        es = jnp.exp(s)
        denom = jnp.dot(es.reshape(n_tok, S), ones_col,
                        preferred_element_type=jnp.float32)
        oh = jax.lax.dot_general(
            es, vh, (((2,), (1,)), ((0,), (0,))),
            preferred_element_type=jnp.float32)
        ohs.append(oh.reshape(n_tok, _DH) / denom)
    x = x + bo[...] + jnp.dot(jnp.concatenate(ohs, axis=1), woc2[...],
                              preferred_element_type=jnp.float32)
    hid = _silu(jnp.dot(_rms_center(x), w1f[...],
                        preferred_element_type=jnp.float32) + bf1f[...])
    x = x + jnp.dot(hid, w2f[...], preferred_element_type=jnp.float32) + bf2[...]
    y = _silu(jnp.dot(x, w3[...], preferred_element_type=jnp.float32)
              + b3c[...])
    o_ref[...] = y.reshape(B, P, S, 8)


def _tail_body(cat_ref, m4, b4, wl, bl, wh, bh, o_ref):
    B, H, C = cat_ref.shape
    c4 = _conv3x3(cat_ref[...], m4[...]).reshape(B * H, 64)
    y = _silu(c4 + b4[...])
    z = _silu(jnp.dot(y, wl[...], preferred_element_type=jnp.float32)
              + bl[...])
    pooled = jnp.mean(z.reshape(B, H, 128), axis=1)
    logits = jnp.dot(pooled, wh[...], preferred_element_type=jnp.float32) \
        + bh[...]
    mx = jnp.max(logits, axis=-1, keepdims=True)
    ex = jnp.exp(logits - mx)
    o_ref[...] = ex / jnp.sum(ex, axis=-1, keepdims=True)


def _blk(shape, b):
    nd = len(shape)
    return pl.BlockSpec((b,) + tuple(shape[1:]),
                        lambda i: (i,) + (0,) * (nd - 1))


def _full(shape):
    nd = len(shape)
    return pl.BlockSpec(tuple(shape), lambda i: (0,) * nd)


def _pick_b(n, want):
    b = min(want, n)
    while n % b:
        b //= 2
    return b


def kernel(x, sd, su, stem_w, stem_a, stem_b, exp_w, exp_a, exp_b, dw_m, dw_a,
           dw_b, proj_w, proj_a, proj_b, c1_m, c1_a, c1_b, c2_w, c2_a, c2_b,
           c3_w, c3_a, c3_b, c4_m, c4_a, c4_b, last_w, last_a, last_b, head_w,
           head_b, l0_g1, l0_b1, l0_wq, l0_wk, l0_wv, l0_wo, l0_bo, l0_g2,
           l0_b2, l0_w1, l0_bf1, l0_w2, l0_bf2):
    n = x.shape[0]
    xh = jnp.transpose(x, (0, 2, 3, 1))

    xp = jnp.pad(xh, ((0, 0), (1, 1), (1, 1), (0, 0)))
    taps = [xp[:, i:i + 16:2, j:j + 16:2, :] for i in range(3) for j in range(3)]
    patches = jnp.concatenate(taps, axis=-1).reshape(n, _W, _W * 27)

    w0 = stem_w * stem_a
    we = exp_w * exp_a
    mdw = dw_m * dw_a
    wpr = proj_w * proj_a
    m1 = c1_m * c1_a
    w2c = c2_w * c2_a
    w3 = c3_w * c3_a
    m4 = c4_m * c4_a
    wl = last_w * last_a

    bf = _pick_b(n, 512)
    y0, c2 = pl.pallas_call(
        _front_body, grid=(n // bf,),
        in_specs=[_blk(patches.shape, bf)] + [
            _full(a.shape) for a in
            (w0, stem_b, we, exp_b, mdw, dw_b, wpr, proj_b, m1, c1_b, w2c, c2_b)
        ],
        out_specs=(_blk((n, _W, 64), bf), _blk((n, _W, 128), bf)),
        out_shape=(jax.ShapeDtypeStruct((n, _W, 64), jnp.float32),
                   jax.ShapeDtypeStruct((n, _W, 128), jnp.float32)),
        compiler_params=_PAR,
    )(patches, w0, stem_b, we, exp_b, mdw, dw_b, wpr, proj_b, m1, c1_b,
      w2c, c2_b)

    ph = pw = 2
    hh = ww = _W // 2
    z = c2.reshape(n, _W, _W, _DIM)
    t = z.reshape(n, hh, ph, ww, pw, _DIM).transpose(0, 2, 4, 1, 3, 5)
    t = t.reshape(n, ph * pw, hh * ww, _DIM)

    scale = _DH ** -0.5
    wqc = jnp.transpose(l0_wq, (1, 0, 2)).reshape(_DIM, _HEADS * _DH) * scale
    wkc = jnp.transpose(l0_wk, (1, 0, 2)).reshape(_DIM, _HEADS * _DH)
    wvc = jnp.transpose(l0_wv, (1, 0, 2)).reshape(_DIM, _HEADS * _DH)
    wqkv = jnp.concatenate([wqc, wkc, wvc], axis=1)
    wqkv = wqkv * l0_g1.reshape(_DIM, 1)
    bqkv = jnp.dot(l0_b1, jnp.concatenate([wqc, wkc, wvc], axis=1))
    wof = l0_wo.reshape(_HEADS * _DH, _DIM)
    w1f = l0_w1 * l0_g2.reshape(_DIM, 1)
    bf1f = l0_bf1 + jnp.dot(l0_b2, l0_w1)

    bt = _pick_b(n, 128)
    t_out = pl.pallas_call(
        _tr_body, grid=(n // bt,),
        in_specs=[_blk(t.shape, bt)] + [
            _full(a.shape) for a in
            (wqkv, bqkv, wof, l0_bo, w1f, bf1f, l0_w2, l0_bf2, w3, c3_b)
        ],
        out_specs=_blk((n, ph * pw, hh * ww, 8), bt),
        out_shape=jax.ShapeDtypeStruct((n, ph * pw, hh * ww, 8), jnp.float32),
        compiler_params=_PAR,
    )(t, wqkv, bqkv, wof, l0_bo, w1f, bf1f, l0_w2, l0_bf2, w3, c3_b)

    c3s = t_out.reshape(n, ph, pw, hh, ww, 8).transpose(0, 3, 1, 4, 2, 5)
    c3s = c3s.reshape(n, _W, _W, 8)
    y0s = y0.reshape(n, _W, _W, 8)
    cat = jnp.concatenate([c3s, y0s], axis=-1).reshape(n, _W, _W * 16)

    bl_ = _pick_b(n, 512)
    probs = pl.pallas_call(
        _tail_body, grid=(n // bl_,),
        in_specs=[_blk(cat.shape, bl_)] + [
            _full(a.shape) for a in (m4, c4_b, wl, last_b, head_w, head_b)
        ],
        out_specs=_blk((n, 5), bl_),
        out_shape=jax.ShapeDtypeStruct((n, 5), jnp.float32),
        compiler_params=_PAR,
    )(cat, m4, c4_b, wl, last_b, head_w, head_b)
    return probs
